# Initial kernel scaffold; baseline (speedup 1.0000x reference)
import jax
import jax.numpy as jnp
from jax.experimental import pallas as pl


def kernel(x, ln1_w, ln2_w, Wq, Wkva, Wkvb, Wo, gateW, expert_bias, gw, uw, dw, sgw, suw, sdw):
    raise NotImplementedError("write your pallas kernel here")


# speedup vs baseline: 1.2033x; 1.2033x over previous
import functools

import numpy as np
import jax
import jax.numpy as jnp
from jax import lax
from jax.experimental import pallas as pl
from jax.experimental.pallas import tpu as pltpu

N_EMBD = 1024
N_HEAD = 16
HEAD_DIM = 64
KV_LORA = 256
ROPE_DIM = 32
NOPE_DIM = HEAD_DIM - ROPE_DIM
N_EXP = 8
TOP_K = 2
INTER = 2048
THETA = 100000.0

BT1 = 256
BTQ = 512
BTE = 256
BTS = 256

HALF = ROPE_DIM // 2


def _rope_tables(T):
    freqs = 1.0 / (THETA ** (np.arange(0, ROPE_DIM, 2, dtype=np.float32) / ROPE_DIM))
    t = np.arange(T, dtype=np.float32)
    f = np.outer(t, freqs)
    return np.cos(f).astype(np.float32), np.sin(f).astype(np.float32)


def _deinterleave_perm():
    p = np.empty((ROPE_DIM,), dtype=np.int32)
    p[:HALF] = 2 * np.arange(HALF)
    p[HALF:] = 2 * np.arange(HALF) + 1
    return p


def _weight_perms():
    pr = _deinterleave_perm()
    qperm = np.arange(N_HEAD * HEAD_DIM, dtype=np.int32)
    for h in range(N_HEAD):
        base = h * HEAD_DIM + NOPE_DIM
        qperm[base:base + ROPE_DIM] = base + pr
    kvaperm = np.arange(KV_LORA + ROPE_DIM, dtype=np.int32)
    kvaperm[KV_LORA:] = KV_LORA + pr
    return qperm, kvaperm


def _k1_body(x_ref, ln1_ref, wq_ref, wkva_ref, wkvb_ref, cos_ref, sin_ref,
             q_ref, knope_ref, krope_ref):
    x = x_ref[...]
    xn = x * lax.rsqrt(jnp.mean(x * x, axis=1, keepdims=True) + 1e-6) * ln1_ref[...]
    q = jnp.dot(xn, wq_ref[...], preferred_element_type=jnp.float32)
    ckv = jnp.dot(xn, wkva_ref[...], preferred_element_type=jnp.float32)
    latent = ckv[:, :KV_LORA]
    kr = ckv[:, KV_LORA:]
    knope_ref[...] = jnp.dot(latent, wkvb_ref[...], preferred_element_type=jnp.float32)
    cos = cos_ref[...]
    sin = sin_ref[...]
    q_ref[...] = q
    for h in range(N_HEAD):
        base = h * HEAD_DIM + NOPE_DIM
        a = q[:, base:base + HALF]
        b = q[:, base + HALF:base + ROPE_DIM]
        q_ref[:, base:base + HALF] = a * cos - b * sin
        q_ref[:, base + HALF:base + ROPE_DIM] = a * sin + b * cos
    a = kr[:, :HALF]
    b = kr[:, HALF:]
    krope_ref[...] = jnp.concatenate([a * cos - b * sin, a * sin + b * cos], axis=1)


def _proj(xf, ln1_w, Wq_p, Wkva_p, Wkvb, cosT, sinT):
    T = xf.shape[0]
    nb = T // BT1
    return pl.pallas_call(
        _k1_body,
        grid=(nb,),
        in_specs=[
            pl.BlockSpec((BT1, N_EMBD), lambda i: (i, 0)),
            pl.BlockSpec((1, N_EMBD), lambda i: (0, 0)),
            pl.BlockSpec((N_EMBD, N_HEAD * HEAD_DIM), lambda i: (0, 0)),
            pl.BlockSpec((N_EMBD, KV_LORA + ROPE_DIM), lambda i: (0, 0)),
            pl.BlockSpec((KV_LORA, N_HEAD * NOPE_DIM), lambda i: (0, 0)),
            pl.BlockSpec((BT1, HALF), lambda i: (i, 0)),
            pl.BlockSpec((BT1, HALF), lambda i: (i, 0)),
        ],
        out_specs=[
            pl.BlockSpec((BT1, N_HEAD * HEAD_DIM), lambda i: (i, 0)),
            pl.BlockSpec((BT1, N_HEAD * NOPE_DIM), lambda i: (i, 0)),
            pl.BlockSpec((BT1, ROPE_DIM), lambda i: (i, 0)),
        ],
        out_shape=[
            jax.ShapeDtypeStruct((T, N_HEAD * HEAD_DIM), jnp.float32),
            jax.ShapeDtypeStruct((T, N_HEAD * NOPE_DIM), jnp.float32),
            jax.ShapeDtypeStruct((T, ROPE_DIM), jnp.float32),
        ],
    )(xf, ln1_w, Wq_p, Wkva_p, Wkvb, cosT, sinT)


def _att_body(q_ref, knope_ref, krope_ref, y_ref):
    q = q_ref[0]
    k = jnp.concatenate([knope_ref[0], krope_ref[...]], axis=1)
    s = lax.dot_general(q, k, (((1,), (1,)), ((), ())),
                        preferred_element_type=jnp.float32) * (1.0 / 8.0)
    i = pl.program_id(1)
    Tk = k.shape[0]
    row = i * BTQ + lax.broadcasted_iota(jnp.int32, (BTQ, Tk), 0)
    col = lax.broadcasted_iota(jnp.int32, (BTQ, Tk), 1)
    s = jnp.where(col <= row, s, jnp.float32(-1e9))
    m = jnp.max(s, axis=1, keepdims=True)
    p = jnp.exp(s - m)
    att = p / jnp.sum(p, axis=1, keepdims=True)
    y_ref[0] = jnp.dot(att, k, preferred_element_type=jnp.float32)


def _attention(q3, knope3, krope):
    T = q3.shape[1]
    nq = T // BTQ
    return pl.pallas_call(
        _att_body,
        grid=(N_HEAD, nq),
        in_specs=[
            pl.BlockSpec((1, BTQ, HEAD_DIM), lambda h, i: (h, i, 0)),
            pl.BlockSpec((1, T, NOPE_DIM), lambda h, i: (h, 0, 0)),
            pl.BlockSpec((T, ROPE_DIM), lambda h, i: (0, 0)),
        ],
        out_specs=pl.BlockSpec((1, BTQ, HEAD_DIM), lambda h, i: (h, i, 0)),
        out_shape=jax.ShapeDtypeStruct((N_HEAD, T, HEAD_DIM), jnp.float32),
    )(q3, knope3, krope)


def _oproj_body(x_ref, y_ref, wo_ref, h_ref):
    h_ref[...] = x_ref[...] + jnp.dot(y_ref[...], wo_ref[...],
                                      preferred_element_type=jnp.float32)


def _oproj(xf, y, Wo_p):
    T = xf.shape[0]
    return pl.pallas_call(
        _oproj_body,
        grid=(T // BT1,),
        in_specs=[
            pl.BlockSpec((BT1, N_EMBD), lambda i: (i, 0)),
            pl.BlockSpec((BT1, N_HEAD * HEAD_DIM), lambda i: (i, 0)),
            pl.BlockSpec((N_HEAD * HEAD_DIM, N_EMBD), lambda i: (0, 0)),
        ],
        out_specs=pl.BlockSpec((BT1, N_EMBD), lambda i: (i, 0)),
        out_shape=jax.ShapeDtypeStruct((T, N_EMBD), jnp.float32),
    )(xf, y, Wo_p)


def _gate_body(h_ref, ln2_ref, gw_ref, bias_ref, xn2_ref, ti_ref, tw_ref):
    h = h_ref[...]
    xn = h * lax.rsqrt(jnp.mean(h * h, axis=1, keepdims=True) + 1e-6) * ln2_ref[...]
    xn2_ref[...] = xn
    logits = jnp.dot(xn, gw_ref[...], preferred_element_type=jnp.float32) + bias_ref[...]
    m = jnp.max(logits, axis=1, keepdims=True)
    e = jnp.exp(logits - m)
    probs = e / jnp.sum(e, axis=1, keepdims=True)
    lane = lax.broadcasted_iota(jnp.int32, probs.shape, 1)
    m1 = jnp.max(probs, axis=1, keepdims=True)
    sel1 = jnp.min(jnp.where(probs == m1, lane, 99), axis=1, keepdims=True)
    p2 = jnp.where(lane == sel1, jnp.float32(-1.0), probs)
    m2 = jnp.max(p2, axis=1, keepdims=True)
    sel2 = jnp.min(jnp.where(p2 == m2, lane, 99), axis=1, keepdims=True)
    denom = m1 + m2
    ti_ref[...] = jnp.where(lane == 0, sel1, jnp.where(lane == 1, sel2, 0))
    tw_ref[...] = jnp.where(lane == 0, m1 / denom,
                            jnp.where(lane == 1, m2 / denom, jnp.float32(0.0)))


def _gate(h, ln2_w, gateW, bias):
    T = h.shape[0]
    return pl.pallas_call(
        _gate_body,
        grid=(T // BT1,),
        in_specs=[
            pl.BlockSpec((BT1, N_EMBD), lambda i: (i, 0)),
            pl.BlockSpec((1, N_EMBD), lambda i: (0, 0)),
            pl.BlockSpec((N_EMBD, N_EXP), lambda i: (0, 0)),
            pl.BlockSpec((1, N_EXP), lambda i: (0, 0)),
        ],
        out_specs=[
            pl.BlockSpec((BT1, N_EMBD), lambda i: (i, 0)),
            pl.BlockSpec((BT1, N_EXP), lambda i: (i, 0)),
            pl.BlockSpec((BT1, N_EXP), lambda i: (i, 0)),
        ],
        out_shape=[
            jax.ShapeDtypeStruct((T, N_EMBD), jnp.float32),
            jax.ShapeDtypeStruct((T, N_EXP), jnp.int32),
            jax.ShapeDtypeStruct((T, N_EXP), jnp.float32),
        ],
    )(h, ln2_w, gateW, bias)


def _expert_body(be_ref, act_ref, xs_ref, sw_ref, gw_ref, uw_ref, dw_ref, ys_ref):
    i = pl.program_id(0)

    @pl.when(act_ref[i] != 0)
    def _():
        x = xs_ref[...]
        g = jnp.dot(x, gw_ref[0], preferred_element_type=jnp.float32)
        u = jnp.dot(x, uw_ref[0], preferred_element_type=jnp.float32)
        a = g * jax.nn.sigmoid(g) * u
        ys_ref[...] = jnp.dot(a, dw_ref[0], preferred_element_type=jnp.float32) * sw_ref[...]


def _experts(be, act, xs, swt, gw, uw, dw):
    NBT = xs.shape[0]
    NB = NBT // BTE
    grid_spec = pltpu.PrefetchScalarGridSpec(
        num_scalar_prefetch=2,
        grid=(NB,),
        in_specs=[
            pl.BlockSpec((BTE, N_EMBD), lambda i, be, act: (i, 0)),
            pl.BlockSpec((BTE, 1), lambda i, be, act: (i, 0)),
            pl.BlockSpec((1, N_EMBD, INTER), lambda i, be, act: (be[i], 0, 0)),
            pl.BlockSpec((1, N_EMBD, INTER), lambda i, be, act: (be[i], 0, 0)),
            pl.BlockSpec((1, INTER, N_EMBD), lambda i, be, act: (be[i], 0, 0)),
        ],
        out_specs=pl.BlockSpec((BTE, N_EMBD), lambda i, be, act: (i, 0)),
    )
    return pl.pallas_call(
        _expert_body,
        grid_spec=grid_spec,
        out_shape=jax.ShapeDtypeStruct((NBT, N_EMBD), jnp.float32),
    )(be, act, xs, swt, gw, uw, dw)


def _combine_body(h_ref, xn2_ref, y0_ref, y1_ref, sgw_ref, suw_ref, sdw_ref, out_ref):
    x = xn2_ref[...]
    g = jnp.dot(x, sgw_ref[...], preferred_element_type=jnp.float32)
    u = jnp.dot(x, suw_ref[...], preferred_element_type=jnp.float32)
    a = g * jax.nn.sigmoid(g) * u
    shared = jnp.dot(a, sdw_ref[...], preferred_element_type=jnp.float32)
    out_ref[...] = h_ref[...] + shared + y0_ref[...] + y1_ref[...]


def _combine(h, xn2, y0, y1, sgw, suw, sdw):
    T = h.shape[0]
    return pl.pallas_call(
        _combine_body,
        grid=(T // BTS,),
        in_specs=[
            pl.BlockSpec((BTS, N_EMBD), lambda i: (i, 0)),
            pl.BlockSpec((BTS, N_EMBD), lambda i: (i, 0)),
            pl.BlockSpec((BTS, N_EMBD), lambda i: (i, 0)),
            pl.BlockSpec((BTS, N_EMBD), lambda i: (i, 0)),
            pl.BlockSpec((N_EMBD, INTER), lambda i: (0, 0)),
            pl.BlockSpec((N_EMBD, INTER), lambda i: (0, 0)),
            pl.BlockSpec((INTER, N_EMBD), lambda i: (0, 0)),
        ],
        out_specs=pl.BlockSpec((BTS, N_EMBD), lambda i: (i, 0)),
        out_shape=jax.ShapeDtypeStruct((T, N_EMBD), jnp.float32),
    )(h, xn2, y0, y1, sgw, suw, sdw)


def _route_meta(ti, tw, T):
    P2 = T * TOP_K
    eid = ti[:, :TOP_K].reshape(-1)
    pw = tw[:, :TOP_K].reshape(-1)
    order = jnp.argsort(eid, stable=True)
    ranks = jnp.zeros((P2,), jnp.int32).at[order].set(
        jnp.arange(P2, dtype=jnp.int32))
    counts = jnp.sum((eid[:, None] == jnp.arange(N_EXP)[None, :]).astype(jnp.int32), axis=0)
    offs = jnp.concatenate([jnp.zeros((1,), jnp.int32), jnp.cumsum(counts)[:-1]])
    nblk = (counts + BTE - 1) // BTE
    ends = jnp.cumsum(nblk)
    bstart = ends - nblk
    pos = bstart[eid] * BTE + ranks - offs[eid]
    NB = P2 // BTE + N_EXP
    NBT = NB * BTE
    tok = (jnp.arange(P2, dtype=jnp.int32) // TOP_K)
    sti = jnp.zeros((NBT,), jnp.int32).at[pos].set(tok)
    swt = jnp.zeros((NBT,), jnp.float32).at[pos].set(pw)
    bids = jnp.arange(NB, dtype=jnp.int32)
    be = jnp.searchsorted(ends, bids, side='right').astype(jnp.int32)
    act = (bids < ends[-1]).astype(jnp.int32)
    be = jnp.where(act == 1, be, 0)
    return sti, swt, be, act, pos


def kernel(x, ln1_w, ln2_w, Wq, Wkva, Wkvb, Wo, gateW, expert_bias, gw, uw, dw, sgw, suw, sdw):
    B, T, C = x.shape
    xf = x.reshape(T, C)

    cos_np, sin_np = _rope_tables(T)
    cosT = jnp.asarray(cos_np)
    sinT = jnp.asarray(sin_np)
    qperm, kvaperm = _weight_perms()
    Wq_p = jnp.take(Wq, qperm, axis=1)
    Wkva_p = jnp.take(Wkva, kvaperm, axis=1)
    Wo_p = jnp.take(Wo, qperm, axis=0)

    q, knope, krope = _proj(xf, ln1_w.reshape(1, C), Wq_p, Wkva_p, Wkvb, cosT, sinT)
    q3 = q.reshape(T, N_HEAD, HEAD_DIM).transpose(1, 0, 2)
    knope3 = knope.reshape(T, N_HEAD, NOPE_DIM).transpose(1, 0, 2)
    y3 = _attention(q3, knope3, krope)
    y = y3.transpose(1, 0, 2).reshape(T, N_HEAD * HEAD_DIM)
    h = _oproj(xf, y, Wo_p)

    xn2, ti, tw = _gate(h, ln2_w.reshape(1, C), gateW, expert_bias.reshape(1, N_EXP))
    sti, swt, be, act, pos = _route_meta(ti, tw, T)

    xs = jnp.take(xn2, sti, axis=0)
    ysw = _experts(be, act, xs, swt[:, None], gw, uw, dw)
    posT = pos.reshape(T, TOP_K)
    y0 = jnp.take(ysw, posT[:, 0], axis=0)
    y1 = jnp.take(ysw, posT[:, 1], axis=0)

    out = _combine(h, xn2, y0, y1, sgw, suw, sdw)
    return out.reshape(B, T, C)


# speedup vs baseline: 1.2633x; 1.0499x over previous
import functools

import numpy as np
import jax
import jax.numpy as jnp
from jax import lax
from jax.experimental import pallas as pl
from jax.experimental.pallas import tpu as pltpu

N_EMBD = 1024
N_HEAD = 16
HEAD_DIM = 64
KV_LORA = 256
ROPE_DIM = 32
NOPE_DIM = HEAD_DIM - ROPE_DIM
N_EXP = 8
TOP_K = 2
INTER = 2048
THETA = 100000.0

BT1 = 256
BTQ = 512
BTE = 256
BTS = 256

HALF = ROPE_DIM // 2


def _rope_tables(T):
    freqs = 1.0 / (THETA ** (np.arange(0, ROPE_DIM, 2, dtype=np.float32) / ROPE_DIM))
    t = np.arange(T, dtype=np.float32)
    f = np.outer(t, freqs)
    return np.cos(f).astype(np.float32), np.sin(f).astype(np.float32)


def _deinterleave_perm():
    p = np.empty((ROPE_DIM,), dtype=np.int32)
    p[:HALF] = 2 * np.arange(HALF)
    p[HALF:] = 2 * np.arange(HALF) + 1
    return p


def _weight_perms():
    pr = _deinterleave_perm()
    qperm = np.arange(N_HEAD * HEAD_DIM, dtype=np.int32)
    for h in range(N_HEAD):
        base = h * HEAD_DIM + NOPE_DIM
        qperm[base:base + ROPE_DIM] = base + pr
    kvaperm = np.arange(KV_LORA + ROPE_DIM, dtype=np.int32)
    kvaperm[KV_LORA:] = KV_LORA + pr
    return qperm, kvaperm


def _k1_body(x_ref, ln1_ref, wq_ref, wkva_ref, wkvb_ref, cos_ref, sin_ref,
             q_ref, knope_ref, krope_ref):
    x = x_ref[...]
    xn = x * lax.rsqrt(jnp.mean(x * x, axis=1, keepdims=True) + 1e-6) * ln1_ref[...]
    xnb = xn.astype(jnp.bfloat16)
    q = jnp.dot(xnb, wq_ref[...].astype(jnp.bfloat16),
                preferred_element_type=jnp.float32)
    ckv = jnp.dot(xnb, wkva_ref[...].astype(jnp.bfloat16),
                  preferred_element_type=jnp.float32)
    latent = ckv[:, :KV_LORA]
    kr = ckv[:, KV_LORA:]
    knope_ref[...] = jnp.dot(latent.astype(jnp.bfloat16),
                             wkvb_ref[...].astype(jnp.bfloat16),
                             preferred_element_type=jnp.float32)
    cos = cos_ref[...]
    sin = sin_ref[...]
    q_ref[...] = q
    for h in range(N_HEAD):
        base = h * HEAD_DIM + NOPE_DIM
        a = q[:, base:base + HALF]
        b = q[:, base + HALF:base + ROPE_DIM]
        q_ref[:, base:base + HALF] = a * cos - b * sin
        q_ref[:, base + HALF:base + ROPE_DIM] = a * sin + b * cos
    a = kr[:, :HALF]
    b = kr[:, HALF:]
    krope_ref[...] = jnp.concatenate([a * cos - b * sin, a * sin + b * cos], axis=1)


def _proj(xf, ln1_w, Wq_p, Wkva_p, Wkvb, cosT, sinT):
    T = xf.shape[0]
    nb = T // BT1
    return pl.pallas_call(
        _k1_body,
        grid=(nb,),
        in_specs=[
            pl.BlockSpec((BT1, N_EMBD), lambda i: (i, 0)),
            pl.BlockSpec((1, N_EMBD), lambda i: (0, 0)),
            pl.BlockSpec((N_EMBD, N_HEAD * HEAD_DIM), lambda i: (0, 0)),
            pl.BlockSpec((N_EMBD, KV_LORA + ROPE_DIM), lambda i: (0, 0)),
            pl.BlockSpec((KV_LORA, N_HEAD * NOPE_DIM), lambda i: (0, 0)),
            pl.BlockSpec((BT1, HALF), lambda i: (i, 0)),
            pl.BlockSpec((BT1, HALF), lambda i: (i, 0)),
        ],
        out_specs=[
            pl.BlockSpec((BT1, N_HEAD * HEAD_DIM), lambda i: (i, 0)),
            pl.BlockSpec((BT1, N_HEAD * NOPE_DIM), lambda i: (i, 0)),
            pl.BlockSpec((BT1, ROPE_DIM), lambda i: (i, 0)),
        ],
        out_shape=[
            jax.ShapeDtypeStruct((T, N_HEAD * HEAD_DIM), jnp.float32),
            jax.ShapeDtypeStruct((T, N_HEAD * NOPE_DIM), jnp.float32),
            jax.ShapeDtypeStruct((T, ROPE_DIM), jnp.float32),
        ],
    )(xf, ln1_w, Wq_p, Wkva_p, Wkvb, cosT, sinT)


def _att_body(q_ref, knope_ref, krope_ref, y_ref):
    q = q_ref[0].astype(jnp.bfloat16)
    k = jnp.concatenate([knope_ref[0], krope_ref[...]],
                        axis=1).astype(jnp.bfloat16)
    s = lax.dot_general(q, k, (((1,), (1,)), ((), ())),
                        preferred_element_type=jnp.float32) * (1.0 / 8.0)
    i = pl.program_id(1)
    Tk = k.shape[0]
    row = i * BTQ + lax.broadcasted_iota(jnp.int32, (BTQ, Tk), 0)
    col = lax.broadcasted_iota(jnp.int32, (BTQ, Tk), 1)
    s = jnp.where(col <= row, s, jnp.float32(-1e9))
    m = jnp.max(s, axis=1, keepdims=True)
    p = jnp.exp(s - m)
    att = (p / jnp.sum(p, axis=1, keepdims=True)).astype(jnp.bfloat16)
    y_ref[0] = jnp.dot(att, k, preferred_element_type=jnp.float32)


def _attention(q3, knope3, krope):
    T = q3.shape[1]
    nq = T // BTQ
    return pl.pallas_call(
        _att_body,
        grid=(N_HEAD, nq),
        in_specs=[
            pl.BlockSpec((1, BTQ, HEAD_DIM), lambda h, i: (h, i, 0)),
            pl.BlockSpec((1, T, NOPE_DIM), lambda h, i: (h, 0, 0)),
            pl.BlockSpec((T, ROPE_DIM), lambda h, i: (0, 0)),
        ],
        out_specs=pl.BlockSpec((1, BTQ, HEAD_DIM), lambda h, i: (h, i, 0)),
        out_shape=jax.ShapeDtypeStruct((N_HEAD, T, HEAD_DIM), jnp.float32),
    )(q3, knope3, krope)


def _oproj_gate_body(x_ref, y_ref, wo_ref, ln2_ref, gw_ref, bias_ref,
                     h_ref, xn2_ref, ti_ref, tw_ref):
    h = x_ref[...] + jnp.dot(y_ref[...].astype(jnp.bfloat16),
                             wo_ref[...].astype(jnp.bfloat16),
                             preferred_element_type=jnp.float32)
    h_ref[...] = h
    xn = h * lax.rsqrt(jnp.mean(h * h, axis=1, keepdims=True) + 1e-6) * ln2_ref[...]
    xn2_ref[...] = xn
    logits = jnp.dot(xn, gw_ref[...], preferred_element_type=jnp.float32) + bias_ref[...]
    m = jnp.max(logits, axis=1, keepdims=True)
    e = jnp.exp(logits - m)
    probs = e / jnp.sum(e, axis=1, keepdims=True)
    lane = lax.broadcasted_iota(jnp.int32, probs.shape, 1)
    m1 = jnp.max(probs, axis=1, keepdims=True)
    sel1 = jnp.min(jnp.where(probs == m1, lane, 99), axis=1, keepdims=True)
    p2 = jnp.where(lane == sel1, jnp.float32(-1.0), probs)
    m2 = jnp.max(p2, axis=1, keepdims=True)
    sel2 = jnp.min(jnp.where(p2 == m2, lane, 99), axis=1, keepdims=True)
    denom = m1 + m2
    ti_ref[...] = jnp.where(lane == 0, sel1, jnp.where(lane == 1, sel2, 0))
    tw_ref[...] = jnp.where(lane == 0, m1 / denom,
                            jnp.where(lane == 1, m2 / denom, jnp.float32(0.0)))


def _oproj_gate(xf, y, Wo_p, ln2_w, gateW, bias):
    T = xf.shape[0]
    return pl.pallas_call(
        _oproj_gate_body,
        grid=(T // BT1,),
        in_specs=[
            pl.BlockSpec((BT1, N_EMBD), lambda i: (i, 0)),
            pl.BlockSpec((BT1, N_HEAD * HEAD_DIM), lambda i: (i, 0)),
            pl.BlockSpec((N_HEAD * HEAD_DIM, N_EMBD), lambda i: (0, 0)),
            pl.BlockSpec((1, N_EMBD), lambda i: (0, 0)),
            pl.BlockSpec((N_EMBD, N_EXP), lambda i: (0, 0)),
            pl.BlockSpec((1, N_EXP), lambda i: (0, 0)),
        ],
        out_specs=[
            pl.BlockSpec((BT1, N_EMBD), lambda i: (i, 0)),
            pl.BlockSpec((BT1, N_EMBD), lambda i: (i, 0)),
            pl.BlockSpec((BT1, N_EXP), lambda i: (i, 0)),
            pl.BlockSpec((BT1, N_EXP), lambda i: (i, 0)),
        ],
        out_shape=[
            jax.ShapeDtypeStruct((T, N_EMBD), jnp.float32),
            jax.ShapeDtypeStruct((T, N_EMBD), jnp.float32),
            jax.ShapeDtypeStruct((T, N_EXP), jnp.int32),
            jax.ShapeDtypeStruct((T, N_EXP), jnp.float32),
        ],
    )(xf, y, Wo_p, ln2_w, gateW, bias)


def _expert_body(be_ref, act_ref, xs_ref, sw_ref, gw_ref, uw_ref, dw_ref, ys_ref):
    i = pl.program_id(0)

    @pl.when(act_ref[i] != 0)
    def _():
        x = xs_ref[...].astype(jnp.bfloat16)
        g = jnp.dot(x, gw_ref[0].astype(jnp.bfloat16),
                    preferred_element_type=jnp.float32)
        u = jnp.dot(x, uw_ref[0].astype(jnp.bfloat16),
                    preferred_element_type=jnp.float32)
        a = (g * jax.nn.sigmoid(g) * u).astype(jnp.bfloat16)
        ys_ref[...] = jnp.dot(a, dw_ref[0].astype(jnp.bfloat16),
                              preferred_element_type=jnp.float32) * sw_ref[...]


def _experts(be, act, xs, swt, gw, uw, dw):
    NBT = xs.shape[0]
    NB = NBT // BTE
    grid_spec = pltpu.PrefetchScalarGridSpec(
        num_scalar_prefetch=2,
        grid=(NB,),
        in_specs=[
            pl.BlockSpec((BTE, N_EMBD), lambda i, be, act: (i, 0)),
            pl.BlockSpec((BTE, 1), lambda i, be, act: (i, 0)),
            pl.BlockSpec((1, N_EMBD, INTER), lambda i, be, act: (be[i], 0, 0)),
            pl.BlockSpec((1, N_EMBD, INTER), lambda i, be, act: (be[i], 0, 0)),
            pl.BlockSpec((1, INTER, N_EMBD), lambda i, be, act: (be[i], 0, 0)),
        ],
        out_specs=pl.BlockSpec((BTE, N_EMBD), lambda i, be, act: (i, 0)),
    )
    return pl.pallas_call(
        _expert_body,
        grid_spec=grid_spec,
        out_shape=jax.ShapeDtypeStruct((NBT, N_EMBD), jnp.float32),
    )(be, act, xs, swt, gw, uw, dw)


def _combine_body(h_ref, xn2_ref, y0_ref, y1_ref, sgw_ref, suw_ref, sdw_ref, out_ref):
    x = xn2_ref[...].astype(jnp.bfloat16)
    g = jnp.dot(x, sgw_ref[...].astype(jnp.bfloat16),
                preferred_element_type=jnp.float32)
    u = jnp.dot(x, suw_ref[...].astype(jnp.bfloat16),
                preferred_element_type=jnp.float32)
    a = (g * jax.nn.sigmoid(g) * u).astype(jnp.bfloat16)
    shared = jnp.dot(a, sdw_ref[...].astype(jnp.bfloat16),
                     preferred_element_type=jnp.float32)
    out_ref[...] = h_ref[...] + shared + y0_ref[...] + y1_ref[...]


def _combine(h, xn2, y0, y1, sgw, suw, sdw):
    T = h.shape[0]
    return pl.pallas_call(
        _combine_body,
        grid=(T // BTS,),
        in_specs=[
            pl.BlockSpec((BTS, N_EMBD), lambda i: (i, 0)),
            pl.BlockSpec((BTS, N_EMBD), lambda i: (i, 0)),
            pl.BlockSpec((BTS, N_EMBD), lambda i: (i, 0)),
            pl.BlockSpec((BTS, N_EMBD), lambda i: (i, 0)),
            pl.BlockSpec((N_EMBD, INTER), lambda i: (0, 0)),
            pl.BlockSpec((N_EMBD, INTER), lambda i: (0, 0)),
            pl.BlockSpec((INTER, N_EMBD), lambda i: (0, 0)),
        ],
        out_specs=pl.BlockSpec((BTS, N_EMBD), lambda i: (i, 0)),
        out_shape=jax.ShapeDtypeStruct((T, N_EMBD), jnp.float32),
    )(h, xn2, y0, y1, sgw, suw, sdw)


def _route_meta(ti, tw, T):
    P2 = T * TOP_K
    eid = ti[:, :TOP_K].reshape(-1)
    pw = tw[:, :TOP_K].reshape(-1)
    oh = (eid[:, None] == jnp.arange(N_EXP, dtype=jnp.int32)[None, :]).astype(jnp.int32)
    csum = jnp.cumsum(oh, axis=0)
    counts = csum[-1]
    rank_within = jnp.sum(oh * (csum - 1), axis=1)
    nblk = (counts + BTE - 1) // BTE
    ends = jnp.cumsum(nblk)
    bstart = ends - nblk
    pos = jnp.sum(oh * bstart[None, :], axis=1) * BTE + rank_within
    NB = P2 // BTE + N_EXP
    NBT = NB * BTE
    tok = (jnp.arange(P2, dtype=jnp.int32) // TOP_K)
    sti = jnp.zeros((NBT,), jnp.int32).at[pos].set(tok)
    swt = jnp.zeros((NBT,), jnp.float32).at[pos].set(pw)
    bids = jnp.arange(NB, dtype=jnp.int32)
    be = jnp.searchsorted(ends, bids, side='right').astype(jnp.int32)
    act = (bids < ends[-1]).astype(jnp.int32)
    be = jnp.where(act == 1, be, 0)
    return sti, swt, be, act, pos


def kernel(x, ln1_w, ln2_w, Wq, Wkva, Wkvb, Wo, gateW, expert_bias, gw, uw, dw, sgw, suw, sdw):
    B, T, C = x.shape
    xf = x.reshape(T, C)

    cos_np, sin_np = _rope_tables(T)
    cosT = jnp.asarray(cos_np)
    sinT = jnp.asarray(sin_np)
    qperm, kvaperm = _weight_perms()
    Wq_p = jnp.take(Wq, qperm, axis=1)
    Wkva_p = jnp.take(Wkva, kvaperm, axis=1)
    Wo_p = jnp.take(Wo, qperm, axis=0)

    q, knope, krope = _proj(xf, ln1_w.reshape(1, C), Wq_p, Wkva_p, Wkvb, cosT, sinT)
    q3 = q.reshape(T, N_HEAD, HEAD_DIM).transpose(1, 0, 2)
    knope3 = knope.reshape(T, N_HEAD, NOPE_DIM).transpose(1, 0, 2)
    y3 = _attention(q3, knope3, krope)
    y = y3.transpose(1, 0, 2).reshape(T, N_HEAD * HEAD_DIM)
    h, xn2, ti, tw = _oproj_gate(xf, y, Wo_p, ln2_w.reshape(1, C), gateW,
                                 expert_bias.reshape(1, N_EXP))
    sti, swt, be, act, pos = _route_meta(ti, tw, T)

    xs = jnp.take(xn2, sti, axis=0)
    ysw = _experts(be, act, xs, swt[:, None], gw, uw, dw)
    posT = pos.reshape(T, TOP_K)
    y0 = jnp.take(ysw, posT[:, 0], axis=0)
    y1 = jnp.take(ysw, posT[:, 1], axis=0)

    out = _combine(h, xn2, y0, y1, sgw, suw, sdw)
    return out.reshape(B, T, C)


# speedup vs baseline: 1.3477x; 1.0668x over previous
import functools

import numpy as np
import jax
import jax.numpy as jnp
from jax import lax
from jax.experimental import pallas as pl
from jax.experimental.pallas import tpu as pltpu

N_EMBD = 1024
N_HEAD = 16
HEAD_DIM = 64
KV_LORA = 256
ROPE_DIM = 32
NOPE_DIM = HEAD_DIM - ROPE_DIM
N_EXP = 8
TOP_K = 2
INTER = 2048
THETA = 100000.0

BT1 = 256
BTQ = 512
BTE = 256
BTS = 256

HALF = ROPE_DIM // 2


def _rope_tables(T):
    freqs = 1.0 / (THETA ** (np.arange(0, ROPE_DIM, 2, dtype=np.float32) / ROPE_DIM))
    t = np.arange(T, dtype=np.float32)
    f = np.outer(t, freqs)
    return np.cos(f).astype(np.float32), np.sin(f).astype(np.float32)


def _deinterleave_perm():
    p = np.empty((ROPE_DIM,), dtype=np.int32)
    p[:HALF] = 2 * np.arange(HALF)
    p[HALF:] = 2 * np.arange(HALF) + 1
    return p


def _weight_perms():
    pr = _deinterleave_perm()
    qperm = np.arange(N_HEAD * HEAD_DIM, dtype=np.int32)
    for h in range(N_HEAD):
        base = h * HEAD_DIM + NOPE_DIM
        qperm[base:base + ROPE_DIM] = base + pr
    kvaperm = np.arange(KV_LORA + ROPE_DIM, dtype=np.int32)
    kvaperm[KV_LORA:] = KV_LORA + pr
    return qperm, kvaperm


def _k1_body(x_ref, ln1_ref, wq_ref, wkva_ref, wkvb_ref, cos_ref, sin_ref,
             q_ref, knope_ref, krope_ref):
    x = x_ref[...]
    xn = x * lax.rsqrt(jnp.mean(x * x, axis=1, keepdims=True) + 1e-6) * ln1_ref[...]
    xnb = xn.astype(jnp.bfloat16)
    q = jnp.dot(xnb, wq_ref[...].astype(jnp.bfloat16),
                preferred_element_type=jnp.float32)
    ckv = jnp.dot(xnb, wkva_ref[...].astype(jnp.bfloat16),
                  preferred_element_type=jnp.float32)
    latent = ckv[:, :KV_LORA]
    kr = ckv[:, KV_LORA:]
    knope_ref[...] = jnp.dot(latent.astype(jnp.bfloat16),
                             wkvb_ref[...].astype(jnp.bfloat16),
                             preferred_element_type=jnp.float32)
    cos = cos_ref[...]
    sin = sin_ref[...]
    q_ref[...] = q
    for h in range(N_HEAD):
        base = h * HEAD_DIM + NOPE_DIM
        a = q[:, base:base + HALF]
        b = q[:, base + HALF:base + ROPE_DIM]
        q_ref[:, base:base + HALF] = a * cos - b * sin
        q_ref[:, base + HALF:base + ROPE_DIM] = a * sin + b * cos
    a = kr[:, :HALF]
    b = kr[:, HALF:]
    krope_ref[...] = jnp.concatenate([a * cos - b * sin, a * sin + b * cos], axis=1)


def _proj(xf, ln1_w, Wq_p, Wkva_p, Wkvb, cosT, sinT):
    T = xf.shape[0]
    nb = T // BT1
    return pl.pallas_call(
        _k1_body,
        grid=(nb,),
        in_specs=[
            pl.BlockSpec((BT1, N_EMBD), lambda i: (i, 0)),
            pl.BlockSpec((1, N_EMBD), lambda i: (0, 0)),
            pl.BlockSpec((N_EMBD, N_HEAD * HEAD_DIM), lambda i: (0, 0)),
            pl.BlockSpec((N_EMBD, KV_LORA + ROPE_DIM), lambda i: (0, 0)),
            pl.BlockSpec((KV_LORA, N_HEAD * NOPE_DIM), lambda i: (0, 0)),
            pl.BlockSpec((BT1, HALF), lambda i: (i, 0)),
            pl.BlockSpec((BT1, HALF), lambda i: (i, 0)),
        ],
        out_specs=[
            pl.BlockSpec((BT1, N_HEAD * HEAD_DIM), lambda i: (i, 0)),
            pl.BlockSpec((BT1, N_HEAD * NOPE_DIM), lambda i: (i, 0)),
            pl.BlockSpec((BT1, ROPE_DIM), lambda i: (i, 0)),
        ],
        out_shape=[
            jax.ShapeDtypeStruct((T, N_HEAD * HEAD_DIM), jnp.float32),
            jax.ShapeDtypeStruct((T, N_HEAD * NOPE_DIM), jnp.float32),
            jax.ShapeDtypeStruct((T, ROPE_DIM), jnp.float32),
        ],
    )(xf, ln1_w, Wq_p, Wkva_p, Wkvb, cosT, sinT)


def _att_body_iq(iq, q_ref, kT_ref, k_ref, y_ref):
    L = (iq + 1) * BTQ
    q = q_ref[0].astype(jnp.bfloat16)
    kT = kT_ref[0].astype(jnp.bfloat16)
    v = k_ref[0].astype(jnp.bfloat16)
    s_diag = jnp.dot(q, kT[:, L - BTQ:],
                     preferred_element_type=jnp.float32) * 0.125
    row = lax.broadcasted_iota(jnp.int32, (BTQ, BTQ), 0)
    col = lax.broadcasted_iota(jnp.int32, (BTQ, BTQ), 1)
    s_diag = jnp.where(col <= row, s_diag, jnp.float32(-1e9))
    if iq == 0:
        m = jnp.max(s_diag, axis=1, keepdims=True)
        p = jnp.exp(s_diag - m).astype(jnp.bfloat16)
        l = jnp.sum(p.astype(jnp.float32), axis=1, keepdims=True)
        y = jnp.dot(p, v, preferred_element_type=jnp.float32)
    else:
        s_pre = jnp.dot(q, kT[:, :L - BTQ],
                        preferred_element_type=jnp.float32) * 0.125
        m = jnp.maximum(jnp.max(s_pre, axis=1, keepdims=True),
                        jnp.max(s_diag, axis=1, keepdims=True))
        p_pre = jnp.exp(s_pre - m).astype(jnp.bfloat16)
        p_diag = jnp.exp(s_diag - m).astype(jnp.bfloat16)
        l = (jnp.sum(p_pre.astype(jnp.float32), axis=1, keepdims=True)
             + jnp.sum(p_diag.astype(jnp.float32), axis=1, keepdims=True))
        y = (jnp.dot(p_pre, v[:L - BTQ], preferred_element_type=jnp.float32)
             + jnp.dot(p_diag, v[L - BTQ:], preferred_element_type=jnp.float32))
    y_ref[0] = y / l


def _attention(q3, kT3, k3):
    T = q3.shape[1]
    nq = T // BTQ
    pieces = []
    for iq in range(nq):
        L = (iq + 1) * BTQ
        pieces.append(pl.pallas_call(
            functools.partial(_att_body_iq, iq),
            grid=(N_HEAD,),
            in_specs=[
                pl.BlockSpec((1, BTQ, HEAD_DIM), lambda h, iq=iq: (h, iq, 0)),
                pl.BlockSpec((1, HEAD_DIM, L), lambda h: (h, 0, 0)),
                pl.BlockSpec((1, L, HEAD_DIM), lambda h: (h, 0, 0)),
            ],
            out_specs=pl.BlockSpec((1, BTQ, HEAD_DIM), lambda h: (h, 0, 0)),
            out_shape=jax.ShapeDtypeStruct((N_HEAD, BTQ, HEAD_DIM), jnp.float32),
        )(q3, kT3, k3))
    return jnp.concatenate(pieces, axis=1)


def _oproj_gate_body(x_ref, y_ref, wo_ref, ln2_ref, gw_ref, bias_ref,
                     h_ref, xn2_ref, ti_ref, tw_ref):
    h = x_ref[...] + jnp.dot(y_ref[...].astype(jnp.bfloat16),
                             wo_ref[...].astype(jnp.bfloat16),
                             preferred_element_type=jnp.float32)
    h_ref[...] = h
    xn = h * lax.rsqrt(jnp.mean(h * h, axis=1, keepdims=True) + 1e-6) * ln2_ref[...]
    xn2_ref[...] = xn
    logits = jnp.dot(xn, gw_ref[...], preferred_element_type=jnp.float32) + bias_ref[...]
    m = jnp.max(logits, axis=1, keepdims=True)
    e = jnp.exp(logits - m)
    probs = e / jnp.sum(e, axis=1, keepdims=True)
    lane = lax.broadcasted_iota(jnp.int32, probs.shape, 1)
    m1 = jnp.max(probs, axis=1, keepdims=True)
    sel1 = jnp.min(jnp.where(probs == m1, lane, 99), axis=1, keepdims=True)
    p2 = jnp.where(lane == sel1, jnp.float32(-1.0), probs)
    m2 = jnp.max(p2, axis=1, keepdims=True)
    sel2 = jnp.min(jnp.where(p2 == m2, lane, 99), axis=1, keepdims=True)
    denom = m1 + m2
    ti_ref[...] = jnp.where(lane == 0, sel1, jnp.where(lane == 1, sel2, 0))
    tw_ref[...] = jnp.where(lane == 0, m1 / denom,
                            jnp.where(lane == 1, m2 / denom, jnp.float32(0.0)))


def _oproj_gate(xf, y, Wo_p, ln2_w, gateW, bias):
    T = xf.shape[0]
    return pl.pallas_call(
        _oproj_gate_body,
        grid=(T // BT1,),
        in_specs=[
            pl.BlockSpec((BT1, N_EMBD), lambda i: (i, 0)),
            pl.BlockSpec((BT1, N_HEAD * HEAD_DIM), lambda i: (i, 0)),
            pl.BlockSpec((N_HEAD * HEAD_DIM, N_EMBD), lambda i: (0, 0)),
            pl.BlockSpec((1, N_EMBD), lambda i: (0, 0)),
            pl.BlockSpec((N_EMBD, N_EXP), lambda i: (0, 0)),
            pl.BlockSpec((1, N_EXP), lambda i: (0, 0)),
        ],
        out_specs=[
            pl.BlockSpec((BT1, N_EMBD), lambda i: (i, 0)),
            pl.BlockSpec((BT1, N_EMBD), lambda i: (i, 0)),
            pl.BlockSpec((BT1, N_EXP), lambda i: (i, 0)),
            pl.BlockSpec((BT1, N_EXP), lambda i: (i, 0)),
        ],
        out_shape=[
            jax.ShapeDtypeStruct((T, N_EMBD), jnp.float32),
            jax.ShapeDtypeStruct((T, N_EMBD), jnp.float32),
            jax.ShapeDtypeStruct((T, N_EXP), jnp.int32),
            jax.ShapeDtypeStruct((T, N_EXP), jnp.float32),
        ],
    )(xf, y, Wo_p, ln2_w, gateW, bias)


def _expert_body(be_ref, act_ref, xs_ref, sw_ref, gw_ref, uw_ref, dw_ref, ys_ref):
    i = pl.program_id(0)

    @pl.when(act_ref[i] != 0)
    def _():
        x = xs_ref[...].astype(jnp.bfloat16)
        g = jnp.dot(x, gw_ref[0].astype(jnp.bfloat16),
                    preferred_element_type=jnp.float32)
        u = jnp.dot(x, uw_ref[0].astype(jnp.bfloat16),
                    preferred_element_type=jnp.float32)
        a = (g * jax.nn.sigmoid(g) * u).astype(jnp.bfloat16)
        ys_ref[...] = jnp.dot(a, dw_ref[0].astype(jnp.bfloat16),
                              preferred_element_type=jnp.float32) * sw_ref[...]


def _experts(be, act, xs, swt, gw, uw, dw):
    NBT = xs.shape[0]
    NB = NBT // BTE
    grid_spec = pltpu.PrefetchScalarGridSpec(
        num_scalar_prefetch=2,
        grid=(NB,),
        in_specs=[
            pl.BlockSpec((BTE, N_EMBD), lambda i, be, act: (i, 0)),
            pl.BlockSpec((BTE, 1), lambda i, be, act: (i, 0)),
            pl.BlockSpec((1, N_EMBD, INTER), lambda i, be, act: (be[i], 0, 0)),
            pl.BlockSpec((1, N_EMBD, INTER), lambda i, be, act: (be[i], 0, 0)),
            pl.BlockSpec((1, INTER, N_EMBD), lambda i, be, act: (be[i], 0, 0)),
        ],
        out_specs=pl.BlockSpec((BTE, N_EMBD), lambda i, be, act: (i, 0)),
    )
    return pl.pallas_call(
        _expert_body,
        grid_spec=grid_spec,
        out_shape=jax.ShapeDtypeStruct((NBT, N_EMBD), jnp.float32),
    )(be, act, xs, swt, gw, uw, dw)


def _combine_body(h_ref, xn2_ref, y0_ref, y1_ref, sgw_ref, suw_ref, sdw_ref, out_ref):
    x = xn2_ref[...].astype(jnp.bfloat16)
    g = jnp.dot(x, sgw_ref[...].astype(jnp.bfloat16),
                preferred_element_type=jnp.float32)
    u = jnp.dot(x, suw_ref[...].astype(jnp.bfloat16),
                preferred_element_type=jnp.float32)
    a = (g * jax.nn.sigmoid(g) * u).astype(jnp.bfloat16)
    shared = jnp.dot(a, sdw_ref[...].astype(jnp.bfloat16),
                     preferred_element_type=jnp.float32)
    out_ref[...] = h_ref[...] + shared + y0_ref[...] + y1_ref[...]


def _combine(h, xn2, y0, y1, sgw, suw, sdw):
    T = h.shape[0]
    return pl.pallas_call(
        _combine_body,
        grid=(T // BTS,),
        in_specs=[
            pl.BlockSpec((BTS, N_EMBD), lambda i: (i, 0)),
            pl.BlockSpec((BTS, N_EMBD), lambda i: (i, 0)),
            pl.BlockSpec((BTS, N_EMBD), lambda i: (i, 0)),
            pl.BlockSpec((BTS, N_EMBD), lambda i: (i, 0)),
            pl.BlockSpec((N_EMBD, INTER), lambda i: (0, 0)),
            pl.BlockSpec((N_EMBD, INTER), lambda i: (0, 0)),
            pl.BlockSpec((INTER, N_EMBD), lambda i: (0, 0)),
        ],
        out_specs=pl.BlockSpec((BTS, N_EMBD), lambda i: (i, 0)),
        out_shape=jax.ShapeDtypeStruct((T, N_EMBD), jnp.float32),
    )(h, xn2, y0, y1, sgw, suw, sdw)


def _route_meta(ti, tw, T):
    P2 = T * TOP_K
    eid = ti[:, :TOP_K].reshape(-1)
    pw = tw[:, :TOP_K].reshape(-1)
    oh = (eid[:, None] == jnp.arange(N_EXP, dtype=jnp.int32)[None, :]).astype(jnp.int32)
    csum = jnp.cumsum(oh, axis=0)
    counts = csum[-1]
    rank_within = jnp.sum(oh * (csum - 1), axis=1)
    nblk = (counts + BTE - 1) // BTE
    ends = jnp.cumsum(nblk)
    bstart = ends - nblk
    pos = jnp.sum(oh * bstart[None, :], axis=1) * BTE + rank_within
    NB = P2 // BTE + N_EXP
    NBT = NB * BTE
    tok = (jnp.arange(P2, dtype=jnp.int32) // TOP_K)
    sti = jnp.zeros((NBT,), jnp.int32).at[pos].set(tok)
    swt = jnp.zeros((NBT,), jnp.float32).at[pos].set(pw)
    bids = jnp.arange(NB, dtype=jnp.int32)
    be = jnp.searchsorted(ends, bids, side='right').astype(jnp.int32)
    act = (bids < ends[-1]).astype(jnp.int32)
    be = jnp.where(act == 1, be, 0)
    return sti, swt, be, act, pos


def kernel(x, ln1_w, ln2_w, Wq, Wkva, Wkvb, Wo, gateW, expert_bias, gw, uw, dw, sgw, suw, sdw):
    B, T, C = x.shape
    xf = x.reshape(T, C)

    cos_np, sin_np = _rope_tables(T)
    cosT = jnp.asarray(cos_np)
    sinT = jnp.asarray(sin_np)
    qperm, kvaperm = _weight_perms()
    Wq_p = jnp.take(Wq, qperm, axis=1)
    Wkva_p = jnp.take(Wkva, kvaperm, axis=1)
    Wo_p = jnp.take(Wo, qperm, axis=0)

    q, knope, krope = _proj(xf, ln1_w.reshape(1, C), Wq_p, Wkva_p, Wkvb, cosT, sinT)
    q3 = q.reshape(T, N_HEAD, HEAD_DIM).transpose(1, 0, 2)
    knope3 = knope.reshape(T, N_HEAD, NOPE_DIM).transpose(1, 0, 2)
    k3 = jnp.concatenate(
        [knope3, jnp.broadcast_to(krope[None], (N_HEAD, T, ROPE_DIM))], axis=2)
    kT3 = k3.transpose(0, 2, 1)
    y3 = _attention(q3, kT3, k3)
    y = y3.transpose(1, 0, 2).reshape(T, N_HEAD * HEAD_DIM)
    h, xn2, ti, tw = _oproj_gate(xf, y, Wo_p, ln2_w.reshape(1, C), gateW,
                                 expert_bias.reshape(1, N_EXP))
    sti, swt, be, act, pos = _route_meta(ti, tw, T)

    xs = jnp.take(xn2, sti, axis=0)
    ysw = _experts(be, act, xs, swt[:, None], gw, uw, dw)
    posT = pos.reshape(T, TOP_K)
    y0 = jnp.take(ysw, posT[:, 0], axis=0)
    y1 = jnp.take(ysw, posT[:, 1], axis=0)

    out = _combine(h, xn2, y0, y1, sgw, suw, sdw)
    return out.reshape(B, T, C)
